# baseline (speedup 1.0000x reference)
import jax
import jax.numpy as jnp
from jax import lax
from jax.experimental import pallas as pl
from jax.experimental.pallas import tpu as pltpu
from jax.experimental.pallas import tpu_sc as plsc

VOCAB_SIZE = 1_000_000
N_EMBD = 64
SEQ_LEN = 200
BATCH = 1024

_info = plsc.get_sparse_core_info()
_NC, _NS = _info.num_cores, _info.num_subcores
NW = _NC * _NS
TOK_PER_W = BATCH * SEQ_LEN // NW
CHUNK = 128
NCHUNK = TOK_PER_W // CHUNK
OUTR = CHUNK // 2
NPAIR = 64
NB = 2


def _emb_body(q_hbm, h_hbm, p_hbm, tt2_hbm, pos2_hbm, out2_hbm,
              q_v, h_v, p_v, pos_v, gbuf, sbuf, gsems, ssems):
    cid = lax.axis_index("c")
    sid = lax.axis_index("s")
    wid = sid * _NC + cid

    pltpu.sync_copy(pos2_hbm, pos_v)
    pltpu.sync_copy(q_hbm.at[wid], q_v)
    pltpu.sync_copy(h_hbm.at[wid], h_v)
    pltpu.sync_copy(p_hbm, p_v)

    def gather(j, b):
        pltpu.async_copy(tt2_hbm.at[q_v.at[j]], gbuf.at[b], gsems.at[b])

    def wait_gather(b):
        pltpu.make_async_copy(tt2_hbm.at[q_v.at[0]], gbuf.at[b],
                              gsems.at[b]).wait()

    def store(j, b):
        base = wid * (NCHUNK * OUTR) + j * OUTR
        pltpu.async_copy(sbuf.at[b], out2_hbm.at[pl.ds(base, OUTR)],
                         ssems.at[b])

    def wait_store(b):
        pltpu.make_async_copy(sbuf.at[b], out2_hbm.at[pl.ds(0, OUTR)],
                              ssems.at[b]).wait()

    for b in range(NB):
        gather(b, b)

    def chunk_body(i, carry):
        for b in range(NB):
            j = i * NB + b
            wait_gather(b)

            @pl.when(j >= NB)
            def _():
                wait_store(b)

            def add_body(g, c):
                u0 = g * 8
                hvec = h_v[j, pl.ds(u0 * 2, 16)]
                pvec = p_v[j, pl.ds(u0, 16)]
                for k in range(8):
                    u = u0 + k
                    h0 = hvec[2 * k]
                    h1 = hvec[2 * k + 1]
                    p = pvec[k]
                    for blk in range(N_EMBD // 16):
                        o = blk * 16
                        sbuf[b, u, pl.ds(o, 16)] = (
                            gbuf[b, 2 * u, pl.ds(h0 + o, 16)]
                            + pos_v[p, pl.ds(o, 16)])
                        sbuf[b, u, pl.ds(64 + o, 16)] = (
                            gbuf[b, 2 * u + 1, pl.ds(h1 + o, 16)]
                            + pos_v[p, pl.ds(64 + o, 16)])
                return c

            lax.fori_loop(0, NPAIR // 8, add_body, 0)
            store(j, b)

            @pl.when(j + NB < NCHUNK)
            def _():
                gather(j + NB, b)
        return carry

    lax.fori_loop(0, NCHUNK // NB, chunk_body, 0)

    for b in range(NB):
        wait_store(b)


def kernel(x, token_table, position_table):
    xi = x.astype(jnp.int32)
    q4 = (xi >> 1).reshape(NW, NCHUNK, CHUNK)
    h4 = jnp.pad(((xi & 1) * N_EMBD).reshape(NW, NCHUNK, CHUNK),
                 ((0, 0), (0, 0), (0, 16)))
    pos_pair = ((jnp.arange(NCHUNK * CHUNK, dtype=jnp.int32) % SEQ_LEN)
                // 2).reshape(NCHUNK, CHUNK)[:, ::2]
    p4 = jnp.pad(pos_pair, ((0, 0), (0, 16)))
    tt2 = token_table.reshape(VOCAB_SIZE // 2, 2 * N_EMBD)
    pos2 = position_table.reshape(SEQ_LEN // 2, 2 * N_EMBD)
    run = pl.kernel(
        _emb_body,
        out_type=jax.ShapeDtypeStruct((BATCH * SEQ_LEN // 2, 2 * N_EMBD),
                                      jnp.float32),
        mesh=plsc.VectorSubcoreMesh(core_axis_name="c", subcore_axis_name="s"),
        scratch_types=[
            pltpu.VMEM((NCHUNK, CHUNK), jnp.int32),
            pltpu.VMEM((NCHUNK, CHUNK + 16), jnp.int32),
            pltpu.VMEM((NCHUNK, NPAIR + 16), jnp.int32),
            pltpu.VMEM((SEQ_LEN // 2, 2 * N_EMBD), jnp.float32),
            pltpu.VMEM((NB, CHUNK, 2 * N_EMBD), jnp.float32),
            pltpu.VMEM((NB, OUTR, 2 * N_EMBD), jnp.float32),
            pltpu.SemaphoreType.DMA((NB,)),
            pltpu.SemaphoreType.DMA((NB,)),
        ],
        compiler_params=pltpu.CompilerParams(use_tc_tiling_on_sc=True),
    )
    out2 = run(q4, h4, p4, tt2, pos2)
    return out2.reshape(BATCH, SEQ_LEN, N_EMBD)
